# baseline (speedup 1.0000x reference)
import functools

import jax
import jax.numpy as jnp
from jax import lax
from jax.experimental import pallas as pl
from jax.experimental.pallas import tpu as pltpu
from jax.experimental.pallas import tpu_sc as plsc

_B = 4096
_L = 200
_D = 128
_E = 8
_FFN = 512
_OUT = 2

_NC = 2
_NS = 16
_NW = _NC * _NS
_TPW = _B // _NW
_C0 = 104
_C1 = 96


_NBUF = 4

_DW = _D // 2


def _pool_body(x_hbm, emb_hbm, out_hbm, ids_v, rows_a, rows_b, out_v,
               sem_a, sem_b):
    wid = lax.axis_index("s") * _NC + lax.axis_index("c")
    tok0 = wid * _TPW
    pltpu.sync_copy(x_hbm.at[pl.ds(tok0 * _L, _TPW * _L)], ids_v)

    def copies(i, s):
        base = i * _L
        return (
            pltpu.make_async_copy(emb_hbm.at[ids_v.at[pl.ds(base, _C0)]],
                                  rows_a[s], sem_a[s]),
            pltpu.make_async_copy(
                emb_hbm.at[ids_v.at[pl.ds(base + _C0, _C1)]],
                rows_b[s], sem_b[s]),
        )

    def issue(i, s):
        ca, cb = copies(i, s)
        ca.start()
        cb.start()

    def acc_rows(rows_ref, n, accs):
        def rbody(r, accs):
            for u in range(4):
                accs = tuple(
                    accs[d] + plsc.bitcast(
                        rows_ref[r * 4 + u, pl.ds(d * 16, 16)], jnp.bfloat16)
                    for d in range(4))
            return accs
        return lax.fori_loop(0, n // 4, rbody, accs)

    for s in range(_NBUF):
        issue(s, s)

    def consume(i, s):
        ca, cb = copies(i, s)
        ca.wait()
        accs = tuple(jnp.zeros((32,), jnp.bfloat16) for _ in range(4))
        accs = acc_rows(rows_a[s], _C0, accs)
        cb.wait()
        accs = acc_rows(rows_b[s], _C1, accs)
        for d in range(4):
            out_v[i, pl.ds(d * 16, 16)] = plsc.bitcast(accs[d], jnp.int32)

    def token_body(j, carry):
        for s in range(_NBUF):
            i = j * _NBUF + s
            consume(i, s)

            @pl.when(i + _NBUF < _TPW)
            def _():
                issue(i + _NBUF, s)
        return carry

    _full = _TPW // _NBUF
    lax.fori_loop(0, _full, token_body, 0)
    for s in range(_TPW % _NBUF):
        consume(_full * _NBUF + s, s)
    pltpu.sync_copy(out_v, out_hbm.at[pl.ds(tok0, _TPW)])


@functools.cache
def _pool():
    return pl.kernel(
        _pool_body,
        out_type=jax.ShapeDtypeStruct((_B, _DW), jnp.int32),
        mesh=plsc.VectorSubcoreMesh(core_axis_name="c", subcore_axis_name="s",
                                    num_cores=_NC, num_subcores=_NS),
        scratch_types=[
            pltpu.VMEM((_TPW * _L,), jnp.int32),
            [pltpu.VMEM((_C0, _DW), jnp.int32) for _ in range(_NBUF)],
            [pltpu.VMEM((_C1, _DW), jnp.int32) for _ in range(_NBUF)],
            pltpu.VMEM((_TPW, _DW), jnp.int32),
            [pltpu.SemaphoreType.DMA for _ in range(_NBUF)],
            [pltpu.SemaphoreType.DMA for _ in range(_NBUF)],
        ],
        compiler_params=pltpu.CompilerParams(needs_layout_passes=False,
                                             use_tc_tiling_on_sc=False),
    )


_BLK = 1024
_VB = 5000


_VPW = 100000 // _NW
_CH = 125


def _packsc_body(emb_hbm, out_hbm, rows, outb, sems):
    wid = lax.axis_index("s") * _NC + lax.axis_index("c")
    row0 = wid * _VPW

    def in_copy(c, sl):
        return pltpu.make_async_copy(
            emb_hbm.at[pl.ds((row0 + c * _CH) * _D, _CH * _D)],
            rows[sl], sems[sl])

    in_copy(0, 0).start()

    def chunk_body(c, carry):
        sl = lax.rem(c, 2)
        for slc in range(2):
            @pl.when(sl == slc)
            def _():
                in_copy(c, slc).wait()

                @pl.when(c + 1 < _VPW // _CH)
                def _():
                    in_copy(c + 1, 1 - slc).start()

                def rbody(r, carry2):
                    for d in range(4):
                        va = plsc.bitcast(
                            rows[slc][pl.ds(r * _D + d * 16, 16)], jnp.int32)
                        vb = plsc.bitcast(
                            rows[slc][pl.ds(r * _D + _DW + d * 16, 16)],
                            jnp.int32)
                        ra = ((va >> 16) & 1) + jnp.int32(0x7FFF)
                        rb = ((vb >> 16) & 1) + jnp.int32(0x7FFF)
                        lo = ((va + ra) >> 16) & jnp.int32(0xFFFF)
                        hi = (vb + rb) & jnp.int32(-65536)
                        outb[slc][r, pl.ds(d * 16, 16)] = lo | hi
                    return carry2

                lax.fori_loop(0, _CH, rbody, 0)
                pltpu.sync_copy(
                    outb[slc], out_hbm.at[pl.ds(row0 + c * _CH, _CH)])
        return carry

    lax.fori_loop(0, _VPW // _CH, chunk_body, 0)


@functools.cache
def _packsc():
    return pl.kernel(
        _packsc_body,
        out_type=jax.ShapeDtypeStruct((100000, _DW), jnp.int32),
        mesh=plsc.VectorSubcoreMesh(core_axis_name="c", subcore_axis_name="s",
                                    num_cores=_NC, num_subcores=_NS),
        scratch_types=[
            [pltpu.VMEM((_CH * _D,), jnp.float32) for _ in range(2)],
            [pltpu.VMEM((_CH, _DW), jnp.int32) for _ in range(2)],
            [pltpu.SemaphoreType.DMA for _ in range(2)],
        ],
        compiler_params=pltpu.CompilerParams(needs_layout_passes=False,
                                             use_tc_tiling_on_sc=False),
    )


def _moe_body(emb_ref, wg_ref, w1_ref, b1_ref, w2_ref, b2_ref, wfc_ref,
              bfc_ref, out_ref):
    s = emb_ref[...]
    f_lo = lax.bitcast_convert_type(s << 16, jnp.float32)
    f_hi = lax.bitcast_convert_type(
        s & jnp.int32(-65536), jnp.float32)
    e = jnp.concatenate([f_lo, f_hi], axis=1) * jnp.float32(1.0 / _L)
    logits = jnp.dot(e, wg_ref[...], preferred_element_type=jnp.float32)
    m = jnp.max(logits, axis=-1, keepdims=True)
    gate = 1.0 / jnp.sum(jnp.exp(logits - m), axis=-1, keepdims=True)
    lane = lax.broadcasted_iota(jnp.int32, logits.shape, 1)
    top1 = jnp.min(jnp.where(logits == m, lane, _E), axis=-1, keepdims=True)
    wfc = wfc_ref[...]
    b2fc = jnp.dot(b2_ref[...], wfc, preferred_element_type=jnp.float32)
    e16 = e.astype(jnp.bfloat16)
    acc = jnp.zeros((e.shape[0], _OUT), jnp.float32)
    for ei in range(_E):
        h = jnp.maximum(
            jnp.dot(e16, w1_ref[ei].astype(jnp.bfloat16),
                    preferred_element_type=jnp.float32)
            + b1_ref[ei], 0.0)
        w2fc = jnp.dot(w2_ref[ei], wfc, preferred_element_type=jnp.float32)
        t = jnp.dot(h.astype(jnp.bfloat16), w2fc.astype(jnp.bfloat16),
                    preferred_element_type=jnp.float32) \
            + b2fc[ei:ei + 1, :]
        acc = acc + jnp.where(top1 == ei, t, 0.0)
    y = gate * acc + bfc_ref[...]
    my = jnp.max(y, axis=-1, keepdims=True)
    lse = my + jnp.log(jnp.sum(jnp.exp(y - my), axis=-1, keepdims=True))
    out_ref[...] = y - lse


def _moe(embedded, W_g, W1, b1, W2, b2, W_fc, b_fc, interpret=False):
    return pl.pallas_call(
        _moe_body,
        grid=(_B // _BLK,),
        in_specs=[
            pl.BlockSpec((_BLK, _DW), lambda i: (i, 0)),
            pl.BlockSpec((_D, _E), lambda i: (0, 0)),
            pl.BlockSpec((_E, _D, _FFN), lambda i: (0, 0, 0)),
            pl.BlockSpec((_E, 1, _FFN), lambda i: (0, 0, 0)),
            pl.BlockSpec((_E, _FFN, _D), lambda i: (0, 0, 0)),
            pl.BlockSpec((_E, _D), lambda i: (0, 0)),
            pl.BlockSpec((_D, _OUT), lambda i: (0, 0)),
            pl.BlockSpec((1, _OUT), lambda i: (0, 0)),
        ],
        out_specs=pl.BlockSpec((_BLK, _OUT), lambda i: (i, 0)),
        out_shape=jax.ShapeDtypeStruct((_B, _OUT), jnp.float32),
        compiler_params=pltpu.CompilerParams(
            dimension_semantics=("arbitrary",)),
        interpret=interpret,
    )(embedded, W_g, W1, b1.reshape(_E, 1, _FFN), W2,
      b2, W_fc, b_fc.reshape(1, _OUT))


def kernel(x, emb, W_g, W1, b1, W2, b2, W_fc, b_fc):
    x_flat = x.reshape(-1).astype(jnp.int32)
    packed = _packsc()(emb.reshape(-1))
    sums = _pool()(x_flat, packed)
    return _moe(sums, W_g, W1, b1, W2, b2, W_fc, b_fc)


# speedup vs baseline: 1.1474x; 1.1474x over previous
import functools

import jax
import jax.numpy as jnp
from jax import lax
from jax.experimental import pallas as pl
from jax.experimental.pallas import tpu as pltpu
from jax.experimental.pallas import tpu_sc as plsc

_B = 4096
_L = 200
_D = 128
_E = 8
_FFN = 512
_OUT = 2

_NC = 2
_NS = 16
_NW = _NC * _NS
_TPW = _B // _NW
_C0 = 104
_C1 = 96

_NBUF = 3


def _pool_body(x_hbm, emb_hbm, out_hbm, ids_v, rows_a, rows_b, out_v,
               sem_a, sem_b):
    wid = lax.axis_index("s") * _NC + lax.axis_index("c")
    tok0 = wid * _TPW
    pltpu.sync_copy(x_hbm.at[pl.ds(tok0 * _L, _TPW * _L)], ids_v)

    def copies(i, s):
        base = i * _L
        return (
            pltpu.make_async_copy(emb_hbm.at[ids_v.at[pl.ds(base, _C0)]],
                                  rows_a[s], sem_a[s]),
            pltpu.make_async_copy(
                emb_hbm.at[ids_v.at[pl.ds(base + _C0, _C1)]],
                rows_b[s], sem_b[s]),
        )

    def issue(i, s):
        ca, cb = copies(i, s)
        ca.start()
        cb.start()

    def acc_rows(rows_ref, n, accs):
        def rbody(r, accs):
            for u in range(4):
                accs = tuple(accs[d] + rows_ref[r * 4 + u, pl.ds(d * 16, 16)]
                             for d in range(8))
            return accs
        return lax.fori_loop(0, n // 4, rbody, accs)

    for s in range(_NBUF):
        issue(s, s)

    def consume(i, s):
        ca, cb = copies(i, s)
        ca.wait()
        accs = tuple(jnp.zeros((16,), jnp.float32) for _ in range(8))
        accs = acc_rows(rows_a[s], _C0, accs)
        cb.wait()
        accs = acc_rows(rows_b[s], _C1, accs)
        scale = jnp.float32(1.0 / _L)
        for d in range(8):
            out_v[i, pl.ds(d * 16, 16)] = accs[d] * scale

    def token_body(j, carry):
        for s in range(_NBUF):
            i = j * _NBUF + s
            consume(i, s)

            @pl.when(i + _NBUF < _TPW)
            def _():
                issue(i + _NBUF, s)
        return carry

    _full = _TPW // _NBUF
    lax.fori_loop(0, _full, token_body, 0)
    for s in range(_TPW % _NBUF):
        consume(_full * _NBUF + s, s)
    pltpu.sync_copy(out_v, out_hbm.at[pl.ds(tok0, _TPW)])


@functools.cache
def _pool():
    return pl.kernel(
        _pool_body,
        out_type=jax.ShapeDtypeStruct((_B, _D), jnp.float32),
        mesh=plsc.VectorSubcoreMesh(core_axis_name="c", subcore_axis_name="s",
                                    num_cores=_NC, num_subcores=_NS),
        scratch_types=[
            pltpu.VMEM((_TPW * _L,), jnp.int32),
            [pltpu.VMEM((_C0, _D), jnp.float32) for _ in range(_NBUF)],
            [pltpu.VMEM((_C1, _D), jnp.float32) for _ in range(_NBUF)],
            pltpu.VMEM((_TPW, _D), jnp.float32),
            [pltpu.SemaphoreType.DMA for _ in range(_NBUF)],
            [pltpu.SemaphoreType.DMA for _ in range(_NBUF)],
        ],
    )


_BLK = 1024


def _moe_body(emb_ref, wg_ref, w1_ref, b1_ref, w2_ref, b2_ref, wfc_ref,
              bfc_ref, out_ref):
    e = emb_ref[...]
    logits = jnp.dot(e, wg_ref[...], preferred_element_type=jnp.float32)
    m = jnp.max(logits, axis=-1, keepdims=True)
    gate = 1.0 / jnp.sum(jnp.exp(logits - m), axis=-1, keepdims=True)
    lane = lax.broadcasted_iota(jnp.int32, logits.shape, 1)
    top1 = jnp.min(jnp.where(logits == m, lane, _E), axis=-1, keepdims=True)
    wfc = wfc_ref[...]
    b2fc = jnp.dot(b2_ref[...], wfc, preferred_element_type=jnp.float32)
    e16 = e.astype(jnp.bfloat16)
    acc = jnp.zeros((e.shape[0], _OUT), jnp.float32)
    for ei in range(_E):
        h = jnp.maximum(
            jnp.dot(e16, w1_ref[ei].astype(jnp.bfloat16),
                    preferred_element_type=jnp.float32)
            + b1_ref[ei], 0.0)
        w2fc = jnp.dot(w2_ref[ei], wfc, preferred_element_type=jnp.float32)
        t = jnp.dot(h.astype(jnp.bfloat16), w2fc.astype(jnp.bfloat16),
                    preferred_element_type=jnp.float32) \
            + b2fc[ei:ei + 1, :]
        acc = acc + jnp.where(top1 == ei, t, 0.0)
    y = gate * acc + bfc_ref[...]
    my = jnp.max(y, axis=-1, keepdims=True)
    lse = my + jnp.log(jnp.sum(jnp.exp(y - my), axis=-1, keepdims=True))
    out_ref[...] = y - lse


def _moe(embedded, W_g, W1, b1, W2, b2, W_fc, b_fc, interpret=False):
    return pl.pallas_call(
        _moe_body,
        grid=(_B // _BLK,),
        in_specs=[
            pl.BlockSpec((_BLK, _D), lambda i: (i, 0)),
            pl.BlockSpec((_D, _E), lambda i: (0, 0)),
            pl.BlockSpec((_E, _D, _FFN), lambda i: (0, 0, 0)),
            pl.BlockSpec((_E, 1, _FFN), lambda i: (0, 0, 0)),
            pl.BlockSpec((_E, _FFN, _D), lambda i: (0, 0, 0)),
            pl.BlockSpec((_E, _D), lambda i: (0, 0)),
            pl.BlockSpec((_D, _OUT), lambda i: (0, 0)),
            pl.BlockSpec((1, _OUT), lambda i: (0, 0)),
        ],
        out_specs=pl.BlockSpec((_BLK, _OUT), lambda i: (i, 0)),
        out_shape=jax.ShapeDtypeStruct((_B, _OUT), jnp.float32),
        compiler_params=pltpu.CompilerParams(
            dimension_semantics=("arbitrary",)),
        interpret=interpret,
    )(embedded, W_g, W1, b1.reshape(_E, 1, _FFN), W2,
      b2, W_fc, b_fc.reshape(1, _OUT))


def kernel(x, emb, W_g, W1, b1, W2, b2, W_fc, b_fc):
    x_flat = x.reshape(-1).astype(jnp.int32)
    embedded = _pool()(x_flat, emb)
    return _moe(embedded, W_g, W1, b1, W2, b2, W_fc, b_fc)


# speedup vs baseline: 1.1636x; 1.0141x over previous
import functools

import jax
import jax.numpy as jnp
from jax import lax
from jax.experimental import pallas as pl
from jax.experimental.pallas import tpu as pltpu
from jax.experimental.pallas import tpu_sc as plsc

_B = 4096
_L = 200
_D = 128
_E = 8
_FFN = 512
_OUT = 2

_NC = 2
_NS = 16
_NW = _NC * _NS
_TPW = _B // _NW
_C0 = 104
_C1 = 96

_NBUF = 3


def _pool_body(x_hbm, emb_hbm, out_hbm, ids_v, rows_a, rows_b, out_v,
               sem_a, sem_b):
    wid = lax.axis_index("s") * _NC + lax.axis_index("c")
    tok0 = wid * _TPW
    pltpu.sync_copy(x_hbm.at[pl.ds(tok0 * _L, _TPW * _L)], ids_v)

    def copies(i, s):
        base = i * _L
        return (
            pltpu.make_async_copy(emb_hbm.at[ids_v.at[pl.ds(base, _C0)]],
                                  rows_a[s], sem_a[s]),
            pltpu.make_async_copy(
                emb_hbm.at[ids_v.at[pl.ds(base + _C0, _C1)]],
                rows_b[s], sem_b[s]),
        )

    def issue(i, s):
        ca, cb = copies(i, s)
        ca.start()
        cb.start()

    def acc_rows(rows_ref, n, accs):
        def rbody(r, accs):
            for u in range(4):
                accs = tuple(accs[d] + rows_ref[r * 4 + u, pl.ds(d * 16, 16)]
                             for d in range(8))
            return accs
        return lax.fori_loop(0, n // 4, rbody, accs)

    for s in range(_NBUF):
        issue(s, s)

    def consume(i, s):
        ca, cb = copies(i, s)
        ca.wait()
        accs = tuple(jnp.zeros((16,), jnp.float32) for _ in range(8))
        accs = acc_rows(rows_a[s], _C0, accs)
        cb.wait()
        accs = acc_rows(rows_b[s], _C1, accs)
        scale = jnp.float32(1.0 / _L)
        for d in range(8):
            out_v[i, pl.ds(d * 16, 16)] = accs[d] * scale

    def token_body(j, carry):
        for s in range(_NBUF):
            i = j * _NBUF + s
            consume(i, s)

            @pl.when(i + _NBUF < _TPW)
            def _():
                issue(i + _NBUF, s)
        return carry

    _full = _TPW // _NBUF
    lax.fori_loop(0, _full, token_body, 0)
    for s in range(_TPW % _NBUF):
        consume(_full * _NBUF + s, s)
    pltpu.sync_copy(out_v, out_hbm.at[pl.ds(tok0, _TPW)])


@functools.cache
def _pool():
    return pl.kernel(
        _pool_body,
        out_type=jax.ShapeDtypeStruct((_B, _D), jnp.float32),
        mesh=plsc.VectorSubcoreMesh(core_axis_name="c", subcore_axis_name="s",
                                    num_cores=_NC, num_subcores=_NS),
        scratch_types=[
            pltpu.VMEM((_TPW * _L,), jnp.int32),
            [pltpu.VMEM((_C0, _D), jnp.float32) for _ in range(_NBUF)],
            [pltpu.VMEM((_C1, _D), jnp.float32) for _ in range(_NBUF)],
            pltpu.VMEM((_TPW, _D), jnp.float32),
            [pltpu.SemaphoreType.DMA for _ in range(_NBUF)],
            [pltpu.SemaphoreType.DMA for _ in range(_NBUF)],
        ],
    )


_BLK = 2048


def _moe_body(emb_ref, wg_ref, w1_ref, b1_ref, w2_ref, b2_ref, wfc_ref,
              bfc_ref, out_ref):
    e = emb_ref[...]
    logits = jnp.dot(e, wg_ref[...], preferred_element_type=jnp.float32)
    m = jnp.max(logits, axis=-1, keepdims=True)
    gate = 1.0 / jnp.sum(jnp.exp(logits - m), axis=-1, keepdims=True)
    lane = lax.broadcasted_iota(jnp.int32, logits.shape, 1)
    top1 = jnp.min(jnp.where(logits == m, lane, _E), axis=-1, keepdims=True)
    wfc = wfc_ref[...]
    b2fc = jnp.dot(b2_ref[...], wfc, preferred_element_type=jnp.float32)
    e16 = e.astype(jnp.bfloat16)
    acc = jnp.zeros((e.shape[0], _OUT), jnp.float32)
    for ei in range(_E):
        h = jnp.maximum(
            jnp.dot(e16, w1_ref[ei].astype(jnp.bfloat16),
                    preferred_element_type=jnp.float32)
            + b1_ref[ei], 0.0)
        w2fc = jnp.dot(w2_ref[ei], wfc, preferred_element_type=jnp.float32)
        t = jnp.dot(h.astype(jnp.bfloat16), w2fc.astype(jnp.bfloat16),
                    preferred_element_type=jnp.float32) \
            + b2fc[ei:ei + 1, :]
        acc = acc + jnp.where(top1 == ei, t, 0.0)
    y = gate * acc + bfc_ref[...]
    my = jnp.max(y, axis=-1, keepdims=True)
    lse = my + jnp.log(jnp.sum(jnp.exp(y - my), axis=-1, keepdims=True))
    out_ref[...] = y - lse


def _moe(embedded, W_g, W1, b1, W2, b2, W_fc, b_fc, interpret=False):
    return pl.pallas_call(
        _moe_body,
        grid=(_B // _BLK,),
        in_specs=[
            pl.BlockSpec((_BLK, _D), lambda i: (i, 0)),
            pl.BlockSpec((_D, _E), lambda i: (0, 0)),
            pl.BlockSpec((_E, _D, _FFN), lambda i: (0, 0, 0)),
            pl.BlockSpec((_E, 1, _FFN), lambda i: (0, 0, 0)),
            pl.BlockSpec((_E, _FFN, _D), lambda i: (0, 0, 0)),
            pl.BlockSpec((_E, _D), lambda i: (0, 0)),
            pl.BlockSpec((_D, _OUT), lambda i: (0, 0)),
            pl.BlockSpec((1, _OUT), lambda i: (0, 0)),
        ],
        out_specs=pl.BlockSpec((_BLK, _OUT), lambda i: (i, 0)),
        out_shape=jax.ShapeDtypeStruct((_B, _OUT), jnp.float32),
        compiler_params=pltpu.CompilerParams(
            dimension_semantics=("parallel",)),
        interpret=interpret,
    )(embedded, W_g, W1, b1.reshape(_E, 1, _FFN), W2,
      b2, W_fc, b_fc.reshape(1, _OUT))


def kernel(x, emb, W_g, W1, b1, W2, b2, W_fc, b_fc):
    x_flat = x.reshape(-1).astype(jnp.int32)
    embedded = _pool()(x_flat, emb)
    return _moe(embedded, W_g, W1, b1, W2, b2, W_fc, b_fc)
